# Initial kernel scaffold; baseline (speedup 1.0000x reference)
import jax
import jax.numpy as jnp
from jax.experimental import pallas as pl


def kernel(s_emb, t_emb, idx):
    raise NotImplementedError("write your pallas kernel here")


# speedup vs baseline: 9.5657x; 9.5657x over previous
import jax
import jax.numpy as jnp
from jax.experimental import pallas as pl
from jax.experimental.pallas import tpu as pltpu

_N = 1024
_D = 512
_TOPK = 10
_HALF = 5
_SIGMA = 1.0
_DELTA = 1.0


def _dist_from_gram(x):
    x2 = jnp.sum(x * x, axis=1)
    g = jax.lax.dot_general(
        x, x, (((1,), (1,)), ((), ())), preferred_element_type=jnp.float32
    )
    d2 = jnp.maximum(x2[:, None] + x2[None, :] - 2.0 * g, 0.0)
    return jnp.where(d2 > 1e-12, jnp.sqrt(jnp.maximum(d2, 1e-12)), 0.0)


def _stml_kernel(s_ref, t_ref, idxc_ref, idxr_ref, out_ref):
    n = _N
    s = s_ref[...]
    t = t_ref[...]
    s = s / jnp.maximum(
        jnp.sqrt(jnp.sum(s * s, axis=1, keepdims=True)), 1e-12
    )
    t = t / jnp.maximum(
        jnp.sqrt(jnp.sum(t * t, axis=1, keepdims=True)), 1e-12
    )

    s_dist = _dist_from_gram(s)
    s_dist = s_dist / jnp.mean(s_dist, axis=1, keepdims=True)

    t_dist = _dist_from_gram(t)
    w_p = jnp.exp(-(t_dist * t_dist) / _SIGMA)

    same = idxc_ref[...] == idxr_ref[...]
    work = jnp.where(same, 1.0, w_p)

    col = jax.lax.broadcasted_iota(jnp.int32, (n, n), 1)
    w_nn = jnp.zeros((n, n), jnp.float32)
    w_half = jnp.zeros((n, n), jnp.float32)
    for k in range(_TOPK):
        rowmax = jnp.max(work, axis=1, keepdims=True)
        is_max = work == rowmax
        sel = jnp.min(jnp.where(is_max, col, n), axis=1, keepdims=True)
        onehot = (col == sel).astype(jnp.float32)
        w_nn = w_nn + onehot
        if k < _HALF:
            w_half = w_half + onehot
        work = jnp.where(onehot > 0.0, -jnp.inf, work)

    v = w_nn * w_nn.T
    cnt = jnp.sum(v, axis=1)
    m = jax.lax.dot_general(
        v, v, (((1,), (0,)), ((), ())), preferred_element_type=jnp.float32
    )
    w_c_tilda = jnp.where(
        cnt[:, None] > 0.0, v * m / jnp.maximum(cnt, 1.0)[:, None], 0.0
    )
    w_c_hat = (
        jax.lax.dot_general(
            w_half, w_c_tilda, (((1,), (0,)), ((), ())),
            preferred_element_type=jnp.float32,
        )
        / float(_HALF)
    )
    w_c = (w_c_hat + w_c_hat.T) / 2.0
    w = (w_p + w_c) / 2.0

    row = jax.lax.broadcasted_iota(jnp.int32, (n, n), 0)
    offdiag = (row != col).astype(jnp.float32)
    pull = jnp.maximum(s_dist, 0.0) ** 2 * (w * offdiag)
    push = jnp.maximum(_DELTA - s_dist, 0.0) ** 2 * ((1.0 - w) * offdiag)
    loss = (jnp.sum(pull) + jnp.sum(push)) / float(n * (n - 1))
    out_ref[...] = jnp.reshape(loss, (1, 1))


def kernel(s_emb, t_emb, idx):
    idx_col = idx.reshape(_N, 1)
    idx_row = idx.reshape(1, _N)
    out = pl.pallas_call(
        _stml_kernel,
        out_shape=jax.ShapeDtypeStruct((1, 1), jnp.float32),
    )(s_emb, t_emb, idx_col, idx_row)
    return out[0, 0]


# speedup vs baseline: 11.6298x; 1.2158x over previous
import jax
import jax.numpy as jnp
from jax.experimental import pallas as pl
from jax.experimental.pallas import tpu as pltpu

_N = 1024
_D = 512
_TOPK = 10
_HALF = 5
_SIGMA = 1.0
_DELTA = 1.0


def _dist_from_gram(x):
    x2 = jnp.sum(x * x, axis=1)
    g = jax.lax.dot_general(
        x, x, (((1,), (1,)), ((), ())), preferred_element_type=jnp.float32
    )
    d2 = jnp.maximum(x2[:, None] + x2[None, :] - 2.0 * g, 0.0)
    return jnp.where(d2 > 1e-12, jnp.sqrt(jnp.maximum(d2, 1e-12)), 0.0)


def _stml_kernel(s_ref, t_ref, idxc_ref, idxr_ref, out_ref):
    n = _N
    s = s_ref[...]
    t = t_ref[...]
    s = s / jnp.maximum(
        jnp.sqrt(jnp.sum(s * s, axis=1, keepdims=True)), 1e-12
    )
    t = t / jnp.maximum(
        jnp.sqrt(jnp.sum(t * t, axis=1, keepdims=True)), 1e-12
    )

    s_dist = _dist_from_gram(s)
    s_dist = s_dist / jnp.mean(s_dist, axis=1, keepdims=True)

    x2t = jnp.sum(t * t, axis=1)
    gt = jax.lax.dot_general(
        t, t, (((1,), (1,)), ((), ())), preferred_element_type=jnp.float32
    )
    d2t = jnp.maximum(x2t[:, None] + x2t[None, :] - 2.0 * gt, 0.0)
    tiny = d2t <= 1e-12
    w_p = jnp.where(tiny, 1.0, jnp.exp(-d2t / _SIGMA))

    same = idxc_ref[...] == idxr_ref[...]

    col = jax.lax.broadcasted_iota(jnp.int32, (n, n), 1)
    d2bits = jax.lax.bitcast_convert_type(d2t, jnp.int32)
    prim = jnp.where(same | tiny, 0, d2bits & ~jnp.int32(1023))
    key = prim | col

    big = jnp.int32(2147483647)
    w_nn = jnp.zeros((n, n), jnp.float32)
    w_half = jnp.zeros((n, n), jnp.float32)
    for k in range(_TOPK):
        rowmin = jnp.min(key, axis=1, keepdims=True)
        onehot_b = key == rowmin
        onehot = onehot_b.astype(jnp.float32)
        w_nn = w_nn + onehot
        if k < _HALF:
            w_half = w_half + onehot
        key = jnp.where(onehot_b, big, key)

    v = w_nn * w_nn.T
    cnt = jnp.sum(v, axis=1)
    m = jax.lax.dot_general(
        v, v, (((1,), (0,)), ((), ())), preferred_element_type=jnp.float32
    )
    w_c_tilda = jnp.where(
        cnt[:, None] > 0.0, v * m / jnp.maximum(cnt, 1.0)[:, None], 0.0
    )
    w_c_hat = (
        jax.lax.dot_general(
            w_half, w_c_tilda, (((1,), (0,)), ((), ())),
            preferred_element_type=jnp.float32,
        )
        / float(_HALF)
    )
    w_c = (w_c_hat + w_c_hat.T) / 2.0
    w = (w_p + w_c) / 2.0

    row = jax.lax.broadcasted_iota(jnp.int32, (n, n), 0)
    offdiag = (row != col).astype(jnp.float32)
    pull = jnp.maximum(s_dist, 0.0) ** 2 * (w * offdiag)
    push = jnp.maximum(_DELTA - s_dist, 0.0) ** 2 * ((1.0 - w) * offdiag)
    loss = (jnp.sum(pull) + jnp.sum(push)) / float(n * (n - 1))
    out_ref[...] = jnp.reshape(loss, (1, 1))


def kernel(s_emb, t_emb, idx):
    idx_col = idx.reshape(_N, 1)
    idx_row = idx.reshape(1, _N)
    out = pl.pallas_call(
        _stml_kernel,
        out_shape=jax.ShapeDtypeStruct((1, 1), jnp.float32),
    )(s_emb, t_emb, idx_col, idx_row)
    return out[0, 0]


# speedup vs baseline: 13.0240x; 1.1199x over previous
import jax
import jax.numpy as jnp
from jax.experimental import pallas as pl
from jax.experimental.pallas import tpu as pltpu

_N = 1024
_D = 512
_TOPK = 10
_HALF = 5
_SIGMA = 1.0
_DELTA = 1.0


def _self_d2(x):
    g = jax.lax.dot_general(
        x, x, (((1,), (1,)), ((), ())), preferred_element_type=jnp.float32
    )
    return jnp.maximum(2.0 - 2.0 * g, 0.0)


def _stml_kernel(s_ref, t_ref, idxc_ref, idxr_ref, out_ref):
    n = _N
    s = s_ref[...]
    t = t_ref[...]
    s = s / jnp.maximum(
        jnp.sqrt(jnp.sum(s * s, axis=1, keepdims=True)), 1e-12
    )
    t = t / jnp.maximum(
        jnp.sqrt(jnp.sum(t * t, axis=1, keepdims=True)), 1e-12
    )

    d2s = _self_d2(s)
    s_dist = jnp.where(d2s > 1e-12, jnp.sqrt(jnp.maximum(d2s, 1e-12)), 0.0)
    s_dist = s_dist / jnp.mean(s_dist, axis=1, keepdims=True)

    d2t = _self_d2(t)
    tiny = d2t <= 1e-12
    w_p = jnp.where(tiny, 1.0, jnp.exp(-d2t / _SIGMA))

    same = idxc_ref[...] == idxr_ref[...]

    col = jax.lax.broadcasted_iota(jnp.int32, (n, n), 1)
    d2bits = jax.lax.bitcast_convert_type(d2t, jnp.int32)
    prim = jnp.where(same | tiny, 0, d2bits & ~jnp.int32(1023))
    key = prim | col

    big = jnp.int32(2147483647)
    w_half = jnp.zeros((n, n), jnp.float32)
    for k in range(_TOPK):
        rowmin = jnp.min(key, axis=1, keepdims=True)
        key = jnp.where(key == rowmin, big, key)
        if k == _HALF - 1:
            w_half = (key == big).astype(jnp.float32)
    w_nn = (key == big).astype(jnp.float32)

    v = w_nn * w_nn.T
    cnt = jnp.sum(v, axis=1)
    m = jax.lax.dot_general(
        v, v, (((1,), (0,)), ((), ())), preferred_element_type=jnp.float32
    )
    w_c_tilda = jnp.where(
        cnt[:, None] > 0.0, v * m / jnp.maximum(cnt, 1.0)[:, None], 0.0
    )
    w_c_hat = (
        jax.lax.dot_general(
            w_half, w_c_tilda, (((1,), (0,)), ((), ())),
            preferred_element_type=jnp.float32,
        )
        / float(_HALF)
    )
    w_c = (w_c_hat + w_c_hat.T) / 2.0
    w = (w_p + w_c) / 2.0

    row = jax.lax.broadcasted_iota(jnp.int32, (n, n), 0)
    offdiag = (row != col).astype(jnp.float32)
    pull = jnp.maximum(s_dist, 0.0) ** 2 * (w * offdiag)
    push = jnp.maximum(_DELTA - s_dist, 0.0) ** 2 * ((1.0 - w) * offdiag)
    loss = (jnp.sum(pull) + jnp.sum(push)) / float(n * (n - 1))
    out_ref[...] = jnp.reshape(loss, (1, 1))


def kernel(s_emb, t_emb, idx):
    idx_col = idx.reshape(_N, 1)
    idx_row = idx.reshape(1, _N)
    out = pl.pallas_call(
        _stml_kernel,
        out_shape=jax.ShapeDtypeStruct((1, 1), jnp.float32),
    )(s_emb, t_emb, idx_col, idx_row)
    return out[0, 0]


# speedup vs baseline: 13.5279x; 1.0387x over previous
import jax
import jax.numpy as jnp
from jax.experimental import pallas as pl
from jax.experimental.pallas import tpu as pltpu

_N = 1024
_D = 512
_TOPK = 10
_HALF = 5
_SIGMA = 1.0
_DELTA = 1.0


def _self_d2(x):
    g = jax.lax.dot_general(
        x, x, (((1,), (1,)), ((), ())), preferred_element_type=jnp.float32
    )
    return jnp.maximum(2.0 - 2.0 * g, 0.0)


def _stml_kernel(s_ref, t_ref, idxc_ref, idxr_ref, out_ref):
    n = _N
    s = s_ref[...]
    t = t_ref[...]
    s = s / jnp.maximum(
        jnp.sqrt(jnp.sum(s * s, axis=1, keepdims=True)), 1e-12
    )
    t = t / jnp.maximum(
        jnp.sqrt(jnp.sum(t * t, axis=1, keepdims=True)), 1e-12
    )

    d2s = _self_d2(s)
    s_dist = jnp.where(d2s > 1e-12, jnp.sqrt(jnp.maximum(d2s, 1e-12)), 0.0)
    s_dist = s_dist / jnp.mean(s_dist, axis=1, keepdims=True)

    d2t = _self_d2(t)
    tiny = d2t <= 1e-12
    w_p = jnp.where(tiny, 1.0, jnp.exp(-d2t / _SIGMA))

    same = idxc_ref[...] == idxr_ref[...]

    rowi = jax.lax.broadcasted_iota(jnp.int32, (n, n), 0)
    d2bits = jax.lax.bitcast_convert_type(d2t, jnp.int32)
    keyT = jnp.where(same | tiny, 0, d2bits & ~jnp.int32(1023)) | rowi

    big = jnp.int32(2147483647)
    w_half_t = None
    for k in range(_TOPK):
        colmin = jnp.min(keyT, axis=0, keepdims=True)
        keyT = jnp.where(keyT == colmin, big, keyT)
        if k == _HALF - 1:
            w_half_t = (keyT == big).astype(jnp.float32)
    w_nn_t = (keyT == big).astype(jnp.float32)

    v = w_nn_t.T * w_nn_t
    cnt = jnp.sum(v, axis=0)
    m = jax.lax.dot_general(
        v, v, (((1,), (0,)), ((), ())), preferred_element_type=jnp.float32
    )
    rc = 0.1 / jnp.maximum(cnt, 1.0)
    x_half = jax.lax.dot_general(
        w_half_t, v * m * rc[:, None], (((0,), (0,)), ((), ())),
        preferred_element_type=jnp.float32,
    )

    rp = jnp.maximum(_DELTA - s_dist, 0.0)
    rp2 = rp * rp
    a2 = 0.5 * (s_dist * s_dist - rp2)
    f = rp2 + a2 * w_p + (a2 + a2.T) * x_half
    col = jax.lax.broadcasted_iota(jnp.int32, (n, n), 1)
    loss = jnp.sum(jnp.where(rowi == col, 0.0, f)) / float(n * (n - 1))
    out_ref[...] = jnp.reshape(loss, (1, 1))


def kernel(s_emb, t_emb, idx):
    idx_col = idx.reshape(_N, 1)
    idx_row = idx.reshape(1, _N)
    out = pl.pallas_call(
        _stml_kernel,
        out_shape=jax.ShapeDtypeStruct((1, 1), jnp.float32),
    )(s_emb, t_emb, idx_col, idx_row)
    return out[0, 0]
